# Initial kernel scaffold; baseline (speedup 1.0000x reference)
import jax
import jax.numpy as jnp
from jax.experimental import pallas as pl


def kernel(x, rowptr, col):
    raise NotImplementedError("write your pallas kernel here")


# speedup vs baseline: 97.4297x; 97.4297x over previous
import functools

import jax
import jax.numpy as jnp
from jax import lax
from jax.experimental import pallas as pl
from jax.experimental.pallas import tpu as pltpu
from jax.experimental.pallas import tpu_sc as plsc

N = 10000
E = 320000
D = 128

NC = 2
NS = 16
NW = NC * NS

EDGES_PER_TILE = E // NW
K = 80
NCHUNK = EDGES_PER_TILE // K

RP_PAD = 16384
SEARCH_BITS = (8192, 4096, 2048, 1024, 512, 256, 128, 64, 32, 16, 8, 4, 2, 1)

ACC_ROWS = 10112
ZROWS = ACC_ROWS // NS

_mesh = plsc.VectorSubcoreMesh(core_axis_name="c", subcore_axis_name="s")


@functools.partial(
    pl.kernel,
    out_type=jax.ShapeDtypeStruct((NC, N, D), jnp.float32),
    mesh=_mesh,
    compiler_params=pltpu.CompilerParams(needs_layout_passes=False),
    scratch_types=[
        pltpu.VMEM((K,), jnp.int32),
        pltpu.VMEM((K,), jnp.int32),
        pltpu.VMEM((K, D), jnp.float32),
        pltpu.VMEM((RP_PAD,), jnp.int32),
        pltpu.VMEM_SHARED((ACC_ROWS, D), jnp.float32),
        pltpu.SemaphoreType.DMA,
    ],
)
def _sc_spmm(x_hbm, rp_hbm, col_hbm, z_hbm, out_hbm,
             idx_v, ridx_v, buf_v, rp_v, acc, sem):
    c = lax.axis_index("c")
    s = lax.axis_index("s")
    tid = c * NS + s

    pltpu.sync_copy(rp_hbm, rp_v)

    pltpu.sync_copy(z_hbm, acc.at[pl.ds(s * ZROWS, ZROWS)])
    plsc.subcore_barrier()

    lanes = lax.iota(jnp.int32, 16)

    def chunk_body(i, _):
        e0 = tid * EDGES_PER_TILE + i * K
        pltpu.sync_copy(col_hbm.at[pl.ds(e0, K)], idx_v)
        pltpu.async_copy(x_hbm.at[idx_v], buf_v, sem).wait()
        for g in range(K // 16):
            evec = lanes + (e0 + g * 16)
            cnt = jnp.zeros((16,), jnp.int32)
            for bit in SEARCH_BITS:
                probe = cnt + (bit - 1)
                v = plsc.load_gather(rp_v, [probe])
                cnt = jnp.where(v <= evec, cnt + bit, cnt)
            row = cnt - 1
            ridx = jnp.where((row < 0) | (row >= N), N, row)
            ridx_v[pl.ds(g * 16, 16)] = ridx
        pltpu.sync_copy(buf_v, acc.at[ridx_v], add=True)
        return ()

    lax.fori_loop(0, NCHUNK, chunk_body, ())
    plsc.subcore_barrier()

    rows_out = 624
    pltpu.sync_copy(acc.at[pl.ds(s * rows_out, rows_out)],
                    out_hbm.at[c, pl.ds(s * rows_out, rows_out)])

    @pl.when(s == NS - 1)
    def _tail():
        pltpu.sync_copy(acc.at[pl.ds(NS * rows_out, N - NS * rows_out)],
                        out_hbm.at[c, pl.ds(NS * rows_out, N - NS * rows_out)])


def _add_body(a_ref, b_ref, o_ref):
    o_ref[...] = a_ref[...] + b_ref[...]


_BLK = 1000


def _partial_sum(parts):
    return pl.pallas_call(
        _add_body,
        grid=(N // _BLK,),
        in_specs=[
            pl.BlockSpec((_BLK, D), lambda i: (i, 0)),
            pl.BlockSpec((_BLK, D), lambda i: (i, 0)),
        ],
        out_specs=pl.BlockSpec((_BLK, D), lambda i: (i, 0)),
        out_shape=jax.ShapeDtypeStruct((N, D), jnp.float32),
    )(parts[0], parts[1])


def kernel(x, rowptr, col):
    rp32 = rowptr.astype(jnp.int32)
    col32 = col.astype(jnp.int32)
    rp_pad = jnp.full((RP_PAD,), jnp.iinfo(jnp.int32).max, jnp.int32)
    rp_pad = rp_pad.at[: N + 1].set(rp32)
    zeros = jnp.zeros((ZROWS, D), jnp.float32)
    parts = _sc_spmm(x, rp_pad, col32, zeros)
    return _partial_sum(parts)


# speedup vs baseline: 163.8625x; 1.6819x over previous
import functools

import jax
import jax.numpy as jnp
from jax import lax
from jax.experimental import pallas as pl
from jax.experimental.pallas import tpu as pltpu
from jax.experimental.pallas import tpu_sc as plsc

N = 10000
E = 320000
D = 128

NC = 2
NS = 16
NW = NC * NS

K = 64
W = 4
TOTAL_WINDOWS = E // (K * W)

RP_PAD = 10016
RP_LAST = RP_PAD - 1
SEARCH_BITS = (8192, 4096, 2048, 1024, 512, 256, 128, 64, 32, 16, 8, 4, 2, 1)

ACC_ROWS = 10112
ZROWS = ACC_ROWS // NS

_mesh = plsc.VectorSubcoreMesh(core_axis_name="c", subcore_axis_name="s")


@functools.partial(
    pl.kernel,
    out_type=jax.ShapeDtypeStruct((NC, N, D), jnp.float32),
    mesh=_mesh,
    compiler_params=pltpu.CompilerParams(needs_layout_passes=False),
    scratch_types=[
        [pltpu.VMEM((K,), jnp.int32) for _ in range(W)],
        [pltpu.VMEM((K,), jnp.int32) for _ in range(W)],
        [pltpu.VMEM((K, D), jnp.float32) for _ in range(W)],
        pltpu.VMEM((RP_PAD,), jnp.int32),
        pltpu.VMEM_SHARED((ACC_ROWS, D), jnp.float32),
        pltpu.SemaphoreType.DMA,
        pltpu.SemaphoreType.DMA,
        pltpu.SemaphoreType.DMA,
    ],
)
def _sc_spmm(x_hbm, rp_hbm, col_hbm, z_hbm, out_hbm,
             idxs, ridxs, bufs, rp_v, acc, csem, gsem, ssem):
    c = lax.axis_index("c")
    s = lax.axis_index("s")
    tid = c * NS + s

    pltpu.sync_copy(rp_hbm, rp_v)

    pltpu.sync_copy(z_hbm, acc.at[pl.ds(s * ZROWS, ZROWS)])
    plsc.subcore_barrier()

    lanes = lax.iota(jnp.int32, 16)

    def search(e0, ridx_ref):
        for g in range(K // 16):
            evec = lanes + (e0 + g * 16)
            cnt = jnp.zeros((16,), jnp.int32)
            for bit in SEARCH_BITS:
                probe = jnp.minimum(cnt + (bit - 1), RP_LAST)
                v = plsc.load_gather(rp_v, [probe])
                cnt = jnp.where(v <= evec, cnt + bit, cnt)
            row = cnt - 1
            ridx_ref[pl.ds(g * 16, 16)] = jnp.where((row < 0) | (row >= N), N, row)

    def win_body(w, _):
        e0 = w * (W * K)
        dc = [pltpu.async_copy(col_hbm.at[pl.ds(e0 + k * K, K)], idxs[k], csem)
              for k in range(W)]
        for d in dc:
            d.wait()
        dg = [pltpu.async_copy(x_hbm.at[idxs[k]], bufs[k], gsem)
              for k in range(W)]
        for k in range(W):
            search(e0 + k * K, ridxs[k])
        for d in dg:
            d.wait()
        dsc = [pltpu.async_copy(bufs[k], acc.at[ridxs[k]], ssem, add=True)
               for k in range(W)]
        for d in dsc:
            d.wait()
        return ()

    w_lo = tid * TOTAL_WINDOWS // NW
    w_hi = (tid + 1) * TOTAL_WINDOWS // NW
    lax.fori_loop(w_lo, w_hi, win_body, ())
    plsc.subcore_barrier()

    rows_out = 624
    pltpu.sync_copy(acc.at[pl.ds(s * rows_out, rows_out)],
                    out_hbm.at[c, pl.ds(s * rows_out, rows_out)])

    @pl.when(s == NS - 1)
    def _tail():
        pltpu.sync_copy(acc.at[pl.ds(NS * rows_out, N - NS * rows_out)],
                        out_hbm.at[c, pl.ds(NS * rows_out, N - NS * rows_out)])


def _add_body(a_ref, b_ref, o_ref):
    o_ref[...] = a_ref[...] + b_ref[...]


_BLK = 1000


def _partial_sum(parts):
    return pl.pallas_call(
        _add_body,
        grid=(N // _BLK,),
        in_specs=[
            pl.BlockSpec((_BLK, D), lambda i: (i, 0)),
            pl.BlockSpec((_BLK, D), lambda i: (i, 0)),
        ],
        out_specs=pl.BlockSpec((_BLK, D), lambda i: (i, 0)),
        out_shape=jax.ShapeDtypeStruct((N, D), jnp.float32),
    )(parts[0], parts[1])


def kernel(x, rowptr, col):
    rp32 = rowptr.astype(jnp.int32)
    col32 = col.astype(jnp.int32)
    rp_pad = jnp.full((RP_PAD,), jnp.iinfo(jnp.int32).max, jnp.int32)
    rp_pad = rp_pad.at[: N + 1].set(rp32[: N + 1])
    zeros = jnp.zeros((ZROWS, D), jnp.float32)
    parts = _sc_spmm(x, rp_pad, col32, zeros)
    return _partial_sum(parts)


# speedup vs baseline: 164.9809x; 1.0068x over previous
import functools

import jax
import jax.numpy as jnp
from jax import lax
from jax.experimental import pallas as pl
from jax.experimental.pallas import tpu as pltpu
from jax.experimental.pallas import tpu_sc as plsc

N = 10000
E = 320000
D = 128

NC = 2
NS = 16
NW = NC * NS

K = 64
W = 4
TOTAL_WINDOWS = E // (K * W)

RP_PAD = 10016
RP_LAST = RP_PAD - 1
SEARCH_BITS = (8192, 4096, 2048, 1024, 512, 256, 128, 64, 32, 16, 8, 4, 2, 1)

ACC_ROWS = 10112
ZROWS = ACC_ROWS // NS

_mesh = plsc.VectorSubcoreMesh(core_axis_name="c", subcore_axis_name="s")


@functools.partial(
    pl.kernel,
    out_type=jax.ShapeDtypeStruct((NC, N, D), jnp.float32),
    mesh=_mesh,
    compiler_params=pltpu.CompilerParams(needs_layout_passes=False),
    scratch_types=[
        [pltpu.VMEM((K,), jnp.int32) for _ in range(W)],
        [pltpu.VMEM((K,), jnp.int32) for _ in range(W)],
        [pltpu.VMEM((K, D), jnp.float32) for _ in range(W)],
        pltpu.VMEM((RP_PAD,), jnp.int32),
        pltpu.VMEM_SHARED((ACC_ROWS, D), jnp.float32),
        pltpu.SemaphoreType.DMA,
        pltpu.SemaphoreType.DMA,
        pltpu.SemaphoreType.DMA,
    ],
)
def _sc_spmm(x_hbm, rp_hbm, col_hbm, z_hbm, out_hbm,
             idxs, ridxs, bufs, rp_v, acc, csem, gsem, ssem):
    c = lax.axis_index("c")
    s = lax.axis_index("s")
    tid = c * NS + s

    pltpu.sync_copy(rp_hbm, rp_v)

    pltpu.sync_copy(z_hbm, acc.at[pl.ds(s * ZROWS, ZROWS)])
    plsc.subcore_barrier()

    lanes = lax.iota(jnp.int32, 16)

    def search(e0, ridx_ref):
        for g in range(K // 16):
            evec = lanes + (e0 + g * 16)
            cnt = jnp.zeros((16,), jnp.int32)
            for bit in SEARCH_BITS:
                probe = jnp.minimum(cnt + (bit - 1), RP_LAST)
                v = plsc.load_gather(rp_v, [probe])
                cnt = jnp.where(v <= evec, cnt + bit, cnt)
            row = cnt - 1
            ridx_ref[pl.ds(g * 16, 16)] = jnp.where((row < 0) | (row >= N), N, row)

    H = W // 2
    SA = tuple(range(H))
    SB = tuple(range(H, W))

    def win_body(w, _):
        e0 = w * (W * K)
        dc = [pltpu.async_copy(col_hbm.at[pl.ds(e0 + k * K, K)], idxs[k], csem)
              for k in range(W)]
        for k in SA:
            dc[k].wait()
        dga = [pltpu.async_copy(x_hbm.at[idxs[k]], bufs[k], gsem) for k in SA]
        for k in SA:
            search(e0 + k * K, ridxs[k])
        for d in dga:
            d.wait()
        dsa = [pltpu.async_copy(bufs[k], acc.at[ridxs[k]], ssem, add=True)
               for k in SA]
        for k in SB:
            dc[k].wait()
        dgb = [pltpu.async_copy(x_hbm.at[idxs[k]], bufs[k], gsem) for k in SB]
        for k in SB:
            search(e0 + k * K, ridxs[k])
        for d in dgb:
            d.wait()
        dsb = [pltpu.async_copy(bufs[k], acc.at[ridxs[k]], ssem, add=True)
               for k in SB]
        for d in dsa + dsb:
            d.wait()
        return ()

    w_lo = tid * TOTAL_WINDOWS // NW
    w_hi = (tid + 1) * TOTAL_WINDOWS // NW
    lax.fori_loop(w_lo, w_hi, win_body, ())
    plsc.subcore_barrier()

    rows_out = 624
    pltpu.sync_copy(acc.at[pl.ds(s * rows_out, rows_out)],
                    out_hbm.at[c, pl.ds(s * rows_out, rows_out)])

    @pl.when(s == NS - 1)
    def _tail():
        pltpu.sync_copy(acc.at[pl.ds(NS * rows_out, N - NS * rows_out)],
                        out_hbm.at[c, pl.ds(NS * rows_out, N - NS * rows_out)])


def _add_body(a_ref, b_ref, o_ref):
    o_ref[...] = a_ref[...] + b_ref[...]


_BLK = 1000


def _partial_sum(parts):
    return pl.pallas_call(
        _add_body,
        grid=(N // _BLK,),
        in_specs=[
            pl.BlockSpec((_BLK, D), lambda i: (i, 0)),
            pl.BlockSpec((_BLK, D), lambda i: (i, 0)),
        ],
        out_specs=pl.BlockSpec((_BLK, D), lambda i: (i, 0)),
        out_shape=jax.ShapeDtypeStruct((N, D), jnp.float32),
    )(parts[0], parts[1])


def kernel(x, rowptr, col):
    rp32 = rowptr.astype(jnp.int32)
    col32 = col.astype(jnp.int32)
    rp_pad = jnp.full((RP_PAD,), jnp.iinfo(jnp.int32).max, jnp.int32)
    rp_pad = rp_pad.at[: N + 1].set(rp32[: N + 1])
    zeros = jnp.zeros((ZROWS, D), jnp.float32)
    parts = _sc_spmm(x, rp_pad, col32, zeros)
    return _partial_sum(parts)


# speedup vs baseline: 165.1014x; 1.0007x over previous
import functools

import jax
import jax.numpy as jnp
from jax import lax
from jax.experimental import pallas as pl
from jax.experimental.pallas import tpu as pltpu
from jax.experimental.pallas import tpu_sc as plsc

N = 10000
E = 320000
D = 128

NC = 2
NS = 16
NW = NC * NS

K = 128
W = 2
TOTAL_WINDOWS = E // (K * W)

RP_PAD = 10016
RP_LAST = RP_PAD - 1
SEARCH_BITS = (8192, 4096, 2048, 1024, 512, 256, 128, 64, 32, 16, 8, 4, 2, 1)

ACC_ROWS = 10112
ZROWS = ACC_ROWS // NS

_mesh = plsc.VectorSubcoreMesh(core_axis_name="c", subcore_axis_name="s")


@functools.partial(
    pl.kernel,
    out_type=jax.ShapeDtypeStruct((NC, N, D), jnp.float32),
    mesh=_mesh,
    compiler_params=pltpu.CompilerParams(needs_layout_passes=False),
    scratch_types=[
        [pltpu.VMEM((K,), jnp.int32) for _ in range(W)],
        [pltpu.VMEM((K,), jnp.int32) for _ in range(W)],
        [pltpu.VMEM((K, D), jnp.float32) for _ in range(W)],
        pltpu.VMEM((RP_PAD,), jnp.int32),
        pltpu.VMEM_SHARED((ACC_ROWS, D), jnp.float32),
        pltpu.SemaphoreType.DMA,
        pltpu.SemaphoreType.DMA,
        pltpu.SemaphoreType.DMA,
    ],
)
def _sc_spmm(x_hbm, rp_hbm, col_hbm, z_hbm, out_hbm,
             idxs, ridxs, bufs, rp_v, acc, csem, gsem, ssem):
    c = lax.axis_index("c")
    s = lax.axis_index("s")
    tid = c * NS + s

    pltpu.sync_copy(rp_hbm, rp_v)

    pltpu.sync_copy(z_hbm, acc.at[pl.ds(s * ZROWS, ZROWS)])
    plsc.subcore_barrier()

    lanes = lax.iota(jnp.int32, 16)

    def search(e0, ridx_ref):
        for g in range(K // 16):
            evec = lanes + (e0 + g * 16)
            cnt = jnp.zeros((16,), jnp.int32)
            for bit in SEARCH_BITS:
                probe = jnp.minimum(cnt + (bit - 1), RP_LAST)
                v = plsc.load_gather(rp_v, [probe])
                cnt = jnp.where(v <= evec, cnt + bit, cnt)
            row = cnt - 1
            ridx_ref[pl.ds(g * 16, 16)] = jnp.where((row < 0) | (row >= N), N, row)

    H = W // 2
    SA = tuple(range(H))
    SB = tuple(range(H, W))

    def win_body(w, _):
        e0 = w * (W * K)
        dc = [pltpu.async_copy(col_hbm.at[pl.ds(e0 + k * K, K)], idxs[k], csem)
              for k in range(W)]
        for k in SA:
            dc[k].wait()
        dga = [pltpu.async_copy(x_hbm.at[idxs[k]], bufs[k], gsem) for k in SA]
        for k in SA:
            search(e0 + k * K, ridxs[k])
        for d in dga:
            d.wait()
        dsa = [pltpu.async_copy(bufs[k], acc.at[ridxs[k]], ssem, add=True)
               for k in SA]
        for k in SB:
            dc[k].wait()
        dgb = [pltpu.async_copy(x_hbm.at[idxs[k]], bufs[k], gsem) for k in SB]
        for k in SB:
            search(e0 + k * K, ridxs[k])
        for d in dgb:
            d.wait()
        dsb = [pltpu.async_copy(bufs[k], acc.at[ridxs[k]], ssem, add=True)
               for k in SB]
        for d in dsa + dsb:
            d.wait()
        return ()

    w_lo = tid * TOTAL_WINDOWS // NW
    w_hi = (tid + 1) * TOTAL_WINDOWS // NW
    lax.fori_loop(w_lo, w_hi, win_body, ())
    plsc.subcore_barrier()

    rows_out = 624
    pltpu.sync_copy(acc.at[pl.ds(s * rows_out, rows_out)],
                    out_hbm.at[c, pl.ds(s * rows_out, rows_out)])

    @pl.when(s == NS - 1)
    def _tail():
        pltpu.sync_copy(acc.at[pl.ds(NS * rows_out, N - NS * rows_out)],
                        out_hbm.at[c, pl.ds(NS * rows_out, N - NS * rows_out)])


def _add_body(a_ref, b_ref, o_ref):
    o_ref[...] = a_ref[...] + b_ref[...]


_BLK = 1000


def _partial_sum(parts):
    return pl.pallas_call(
        _add_body,
        grid=(N // _BLK,),
        in_specs=[
            pl.BlockSpec((_BLK, D), lambda i: (i, 0)),
            pl.BlockSpec((_BLK, D), lambda i: (i, 0)),
        ],
        out_specs=pl.BlockSpec((_BLK, D), lambda i: (i, 0)),
        out_shape=jax.ShapeDtypeStruct((N, D), jnp.float32),
    )(parts[0], parts[1])


def kernel(x, rowptr, col):
    rp32 = rowptr.astype(jnp.int32)
    col32 = col.astype(jnp.int32)
    rp_pad = jnp.full((RP_PAD,), jnp.iinfo(jnp.int32).max, jnp.int32)
    rp_pad = rp_pad.at[: N + 1].set(rp32[: N + 1])
    zeros = jnp.zeros((ZROWS, D), jnp.float32)
    parts = _sc_spmm(x, rp_pad, col32, zeros)
    return _partial_sum(parts)


# speedup vs baseline: 205.9656x; 1.2475x over previous
import functools

import jax
import jax.numpy as jnp
from jax import lax
from jax.experimental import pallas as pl
from jax.experimental.pallas import tpu as pltpu
from jax.experimental.pallas import tpu_sc as plsc

N = 10000
E = 320000
D = 128

NC = 2
NS = 16
NW = NC * NS

K = 48
SLOTS = 6
GA = 3
TOTAL_CHUNKS = 6672
CMAX = TOTAL_CHUNKS - 1
E_PAD = TOTAL_CHUNKS * K
TOTAL_HEX = TOTAL_CHUNKS // SLOTS

RP_PAD = 10016
RP_LAST = RP_PAD - 1
SEARCH_BITS = (8192, 4096, 2048, 1024, 512, 256, 128, 64, 32, 16, 8, 4, 2, 1)

ACC_ROWS = 10112
ZROWS = ACC_ROWS // NS

_mesh = plsc.VectorSubcoreMesh(core_axis_name="c", subcore_axis_name="s")


@functools.partial(
    pl.kernel,
    out_type=jax.ShapeDtypeStruct((NC, N, D), jnp.float32),
    mesh=_mesh,
    compiler_params=pltpu.CompilerParams(needs_layout_passes=False),
    scratch_types=[
        [pltpu.VMEM((K,), jnp.int32) for _ in range(SLOTS)],
        [pltpu.VMEM((K,), jnp.int32) for _ in range(SLOTS)],
        [pltpu.VMEM((K, D), jnp.float32) for _ in range(SLOTS)],
        pltpu.VMEM((RP_PAD,), jnp.int32),
        pltpu.VMEM_SHARED((ACC_ROWS, D), jnp.float32),
        pltpu.SemaphoreType.DMA,
        pltpu.SemaphoreType.DMA,
        pltpu.SemaphoreType.DMA,
    ],
)
def _sc_spmm(x_hbm, rp_hbm, col_hbm, z_hbm, out_hbm,
             idxs, ridxs, bufs, rp_v, acc, csem, gsem, ssem):
    c = lax.axis_index("c")
    s = lax.axis_index("s")
    tid = c * NS + s

    pltpu.sync_copy(rp_hbm, rp_v)

    pltpu.sync_copy(z_hbm, acc.at[pl.ds(s * ZROWS, ZROWS)])
    plsc.subcore_barrier()

    lanes = lax.iota(jnp.int32, 16)

    def search(e0, ridx_ref):
        for g in range(K // 16):
            evec = lanes + (e0 + g * 16)
            cnt = jnp.zeros((16,), jnp.int32)
            for bit in SEARCH_BITS:
                probe = jnp.minimum(cnt + (bit - 1), RP_LAST)
                v = plsc.load_gather(rp_v, [probe])
                cnt = jnp.where(v <= evec, cnt + bit, cnt)
            row = cnt - 1
            ridx_ref[pl.ds(g * 16, 16)] = jnp.where((row < 0) | (row >= N), N, row)

    def issue_col(ch, slot):
        off = jnp.minimum(ch, CMAX) * K
        pltpu.async_copy(col_hbm.at[pl.ds(off, K)], idxs[slot], csem)

    def wait_col():
        pltpu.make_async_copy(col_hbm.at[pl.ds(0, K)], idxs[0], csem).wait()

    def wait_buf(sem):
        pltpu.make_async_copy(x_hbm.at[pl.ds(0, K)], bufs[0], sem).wait()

    h_lo = tid * TOTAL_HEX // NW
    h_hi = (tid + 1) * TOTAL_HEX // NW
    c0 = SLOTS * h_lo

    for k in range(GA, SLOTS):
        for g in range(K // 16):
            ridxs[k][pl.ds(g * 16, 16)] = jnp.full((16,), N, jnp.int32)
    for k in range(GA, SLOTS):
        pltpu.async_copy(bufs[k], acc.at[ridxs[k]], ssem, add=True)

    for k in range(SLOTS - 1):
        issue_col(c0 + k, k)
    for _ in range(GA):
        wait_col()
    for k in range(GA):
        pltpu.async_copy(x_hbm.at[idxs[k]], bufs[k], gsem)

    def hex_body(h, _):
        cb = SLOTS * h
        for k in range(SLOTS):
            ch = cb + k
            wait_buf(ssem)
            wait_buf(gsem)
            issue_col(ch + SLOTS - 1, (k + SLOTS - 1) % SLOTS)
            search(ch * K, ridxs[k])
            pltpu.async_copy(bufs[k], acc.at[ridxs[k]], ssem, add=True)
            wait_col()
            pltpu.async_copy(x_hbm.at[idxs[(k + GA) % SLOTS]],
                             bufs[(k + GA) % SLOTS], gsem)
        return ()

    lax.fori_loop(h_lo, h_hi, hex_body, ())

    for _ in range(GA):
        wait_buf(ssem)
        wait_buf(gsem)
    for _ in range(2):
        wait_col()
    plsc.subcore_barrier()

    rows_out = 624
    pltpu.sync_copy(acc.at[pl.ds(s * rows_out, rows_out)],
                    out_hbm.at[c, pl.ds(s * rows_out, rows_out)])

    @pl.when(s == NS - 1)
    def _tail():
        pltpu.sync_copy(acc.at[pl.ds(NS * rows_out, N - NS * rows_out)],
                        out_hbm.at[c, pl.ds(NS * rows_out, N - NS * rows_out)])


def _add_body(a_ref, b_ref, o_ref):
    o_ref[...] = a_ref[...] + b_ref[...]


_BLK = 1000


def _partial_sum(parts):
    return pl.pallas_call(
        _add_body,
        grid=(N // _BLK,),
        in_specs=[
            pl.BlockSpec((_BLK, D), lambda i: (i, 0)),
            pl.BlockSpec((_BLK, D), lambda i: (i, 0)),
        ],
        out_specs=pl.BlockSpec((_BLK, D), lambda i: (i, 0)),
        out_shape=jax.ShapeDtypeStruct((N, D), jnp.float32),
    )(parts[0], parts[1])


def kernel(x, rowptr, col):
    rp32 = rowptr.astype(jnp.int32)
    col32 = col.astype(jnp.int32)
    rp_pad = jnp.full((RP_PAD,), jnp.iinfo(jnp.int32).max, jnp.int32)
    rp_pad = rp_pad.at[: N + 1].set(rp32[: N + 1])
    col_pad = jnp.zeros((E_PAD,), jnp.int32).at[:E].set(col32)
    zeros = jnp.zeros((ZROWS, D), jnp.float32)
    parts = _sc_spmm(x, rp_pad, col_pad, zeros)
    return _partial_sum(parts)
